# Initial kernel scaffold; baseline (speedup 1.0000x reference)
import jax
import jax.numpy as jnp
from jax.experimental import pallas as pl


def kernel(features, edge_index, W1, b1, W2, b2, W3, b3):
    raise NotImplementedError("write your pallas kernel here")


# speedup vs baseline: 3.6427x; 3.6427x over previous
import functools

import jax
import jax.numpy as jnp
from jax import lax
from jax.experimental import pallas as pl
from jax.experimental.pallas import tpu as pltpu
from jax.experimental.pallas import tpu_sc as plsc

N = 10000
E = 320000
D = 128

NC = 2
NS = 16
NW = NC * NS

NPAD = 10240
ROWS_PER_TEC = NPAD // NS
EB = 128
EPT = 10240
NB = EPT // EB
EPAD = NW * EPT

_mesh = plsc.VectorSubcoreMesh(core_axis_name="c", subcore_axis_name="s")


@functools.partial(
    pl.kernel,
    mesh=_mesh,
    out_type=jax.ShapeDtypeStruct((NC, NPAD, D), jnp.float32),
    scratch_types=[
        pltpu.VMEM((EB,), jnp.int32),
        pltpu.VMEM((EB,), jnp.int32),
        pltpu.VMEM((EB, D), jnp.float32),
        pltpu.VMEM((EB, D), jnp.float32),
        pltpu.VMEM_SHARED((NPAD, D), jnp.float32),
    ],
)
def _deg_kernel(src_hbm, dst_hbm, e0_hbm, e1_hbm, zeros_hbm,
                out_hbm,
                src_v, dst_v, e0_v, e1_v, sp):
    c = lax.axis_index("c")
    s = lax.axis_index("s")
    wid = c * NS + s
    rbase = s * ROWS_PER_TEC

    pltpu.sync_copy(zeros_hbm.at[pl.ds(rbase, ROWS_PER_TEC)],
                    sp.at[pl.ds(rbase, ROWS_PER_TEC)])
    pltpu.sync_copy(e0_hbm, e0_v)
    pltpu.sync_copy(e1_hbm, e1_v)
    plsc.subcore_barrier()

    ebase = wid * EPT

    def body(j, carry):
        off = ebase + j * EB
        pltpu.sync_copy(src_hbm.at[pl.ds(off, EB)], src_v)
        pltpu.sync_copy(dst_hbm.at[pl.ds(off, EB)], dst_v)
        pltpu.sync_copy(e0_v, sp.at[src_v], add=True)
        pltpu.sync_copy(e1_v, sp.at[dst_v], add=True)
        return carry

    lax.fori_loop(0, NB, body, 0)
    plsc.subcore_barrier()

    pltpu.sync_copy(sp.at[pl.ds(rbase, ROWS_PER_TEC)],
                    out_hbm.at[c, pl.ds(rbase, ROWS_PER_TEC)])


@functools.partial(
    pl.kernel,
    mesh=_mesh,
    out_type=jax.ShapeDtypeStruct((NC, NPAD, D), jnp.float32),
    scratch_types=[
        pltpu.VMEM((EB,), jnp.int32),
        pltpu.VMEM((EB,), jnp.int32),
        pltpu.VMEM((EB, D), jnp.float32),
        pltpu.VMEM_SHARED((NPAD, D), jnp.float32),
        pltpu.SemaphoreType.DMA,
    ],
)
def _agg_kernel(src_hbm, dst_hbm, hw_hbm, zeros_hbm,
                out_hbm,
                src_v, dst_v, rows_v, agg_sp, sem):
    c = lax.axis_index("c")
    s = lax.axis_index("s")
    wid = c * NS + s
    rbase = s * ROWS_PER_TEC

    pltpu.sync_copy(zeros_hbm.at[pl.ds(rbase, ROWS_PER_TEC)],
                    agg_sp.at[pl.ds(rbase, ROWS_PER_TEC)])
    plsc.subcore_barrier()

    ebase = wid * EPT

    def body(j, carry):
        off = ebase + j * EB
        pltpu.sync_copy(src_hbm.at[pl.ds(off, EB)], src_v)
        pltpu.sync_copy(dst_hbm.at[pl.ds(off, EB)], dst_v)
        pltpu.async_copy(hw_hbm.at[src_v], rows_v, sem).wait()
        pltpu.sync_copy(rows_v, agg_sp.at[dst_v], add=True)
        return carry

    lax.fori_loop(0, NB, body, 0)
    plsc.subcore_barrier()

    pltpu.sync_copy(agg_sp.at[pl.ds(rbase, ROWS_PER_TEC)],
                    out_hbm.at[c, pl.ds(rbase, ROWS_PER_TEC)])


def _norm_body(deg_ref, ns_ref, nd_ref):
    co = deg_ref[0, :, 0:1] + deg_ref[1, :, 0:1]
    ci = deg_ref[0, :, 1:2] + deg_ref[1, :, 1:2]
    ns_ref[:] = lax.rsqrt(co + 1.0)
    nd_ref[:] = lax.rsqrt(ci + 1.0)


def _norms(deg):
    return pl.pallas_call(
        _norm_body,
        out_shape=[
            jax.ShapeDtypeStruct((NPAD, 1), jnp.float32),
            jax.ShapeDtypeStruct((NPAD, 1), jnp.float32),
        ],
    )(deg)


_BLK = 1024


def _pre_body(h_ref, ns_ref, w_ref, o_ref):
    o_ref[:] = jnp.dot(h_ref[:] * ns_ref[:], w_ref[:],
                       preferred_element_type=jnp.float32)


def _pre_matmul(h, ns, w):
    grid = (NPAD // _BLK,)
    return pl.pallas_call(
        _pre_body,
        grid=grid,
        in_specs=[
            pl.BlockSpec((_BLK, D), lambda i: (i, 0)),
            pl.BlockSpec((_BLK, 1), lambda i: (i, 0)),
            pl.BlockSpec((D, D), lambda i: (0, 0)),
        ],
        out_specs=pl.BlockSpec((_BLK, D), lambda i: (i, 0)),
        out_shape=jax.ShapeDtypeStruct((NPAD, D), jnp.float32),
    )(h, ns, w)


def _mid_body(p_ref, hw_ref, nd_ref, b_ref, ns_ref, w_ref, o_ref):
    agg = p_ref[0] + p_ref[1] + hw_ref[:]
    h = jnp.maximum(agg * nd_ref[:] + b_ref[:], 0.0) * ns_ref[:]
    o_ref[:] = jnp.dot(h, w_ref[:], preferred_element_type=jnp.float32)


def _mid_matmul(parts, hw, nd, b, ns, w):
    grid = (NPAD // _BLK,)
    return pl.pallas_call(
        _mid_body,
        grid=grid,
        in_specs=[
            pl.BlockSpec((NC, _BLK, D), lambda i: (0, i, 0)),
            pl.BlockSpec((_BLK, D), lambda i: (i, 0)),
            pl.BlockSpec((_BLK, 1), lambda i: (i, 0)),
            pl.BlockSpec((1, D), lambda i: (0, 0)),
            pl.BlockSpec((_BLK, 1), lambda i: (i, 0)),
            pl.BlockSpec((D, D), lambda i: (0, 0)),
        ],
        out_specs=pl.BlockSpec((_BLK, D), lambda i: (i, 0)),
        out_shape=jax.ShapeDtypeStruct((NPAD, D), jnp.float32),
    )(parts, hw, nd, b, ns, w)


def _fin_body(p_ref, hw_ref, nd_ref, b_ref, o_ref):
    agg = p_ref[0] + p_ref[1] + hw_ref[:]
    o_ref[:] = agg * nd_ref[:] + b_ref[:]


def _fin_combine(parts, hw, nd, b):
    grid = (NPAD // _BLK,)
    return pl.pallas_call(
        _fin_body,
        grid=grid,
        in_specs=[
            pl.BlockSpec((NC, _BLK, D), lambda i: (0, i, 0)),
            pl.BlockSpec((_BLK, D), lambda i: (i, 0)),
            pl.BlockSpec((_BLK, 1), lambda i: (i, 0)),
            pl.BlockSpec((1, D), lambda i: (0, 0)),
        ],
        out_specs=pl.BlockSpec((_BLK, D), lambda i: (i, 0)),
        out_shape=jax.ShapeDtypeStruct((NPAD, D), jnp.float32),
    )(parts, hw, nd, b)


def kernel(features, edge_index, W1, b1, W2, b2, W3, b3):
    src = edge_index[0].astype(jnp.int32)
    dst = edge_index[1].astype(jnp.int32)
    pad = EPAD - E
    fill = jnp.full((pad,), NPAD - 1, jnp.int32)
    srcp = jnp.concatenate([src, fill])
    dstp = jnp.concatenate([dst, fill])

    featp = jnp.pad(features, ((0, NPAD - N), (0, 0)))
    zerosD = jnp.zeros((NPAD, D), jnp.float32)
    eye2 = jnp.eye(2, D, dtype=jnp.float32)
    e0 = jnp.broadcast_to(eye2[0:1], (EB, D))
    e1 = jnp.broadcast_to(eye2[1:2], (EB, D))

    deg = _deg_kernel(srcp, dstp, e0, e1, zerosD)
    ns, nd = _norms(deg)

    b1r = b1.reshape(1, D)
    b2r = b2.reshape(1, D)
    b3r = b3.reshape(1, D)

    hw = _pre_matmul(featp, ns, W1)
    parts = _agg_kernel(srcp, dstp, hw, zerosD)
    hw = _mid_matmul(parts, hw, nd, b1r, ns, W2)
    parts = _agg_kernel(srcp, dstp, hw, zerosD)
    hw = _mid_matmul(parts, hw, nd, b2r, ns, W2)
    parts = _agg_kernel(srcp, dstp, hw, zerosD)
    hw = _mid_matmul(parts, hw, nd, b2r, ns, W3)
    parts = _agg_kernel(srcp, dstp, hw, zerosD)
    out = _fin_combine(parts, hw, nd, b3r)

    return out[:N]


# speedup vs baseline: 4.5005x; 1.2355x over previous
import functools

import jax
import jax.numpy as jnp
from jax import lax
from jax.experimental import pallas as pl
from jax.experimental.pallas import tpu as pltpu
from jax.experimental.pallas import tpu_sc as plsc

N = 10000
E = 320000
D = 128

NC = 2
NS = 16
NW = NC * NS

NPAD = 10240
ROWS_PER_TEC = NPAD // NS
EB = 128
EPT = 10240
NB = EPT // EB
EPAD = NW * EPT

_mesh = plsc.VectorSubcoreMesh(core_axis_name="c", subcore_axis_name="s")


@functools.partial(
    pl.kernel,
    mesh=_mesh,
    out_type=jax.ShapeDtypeStruct((NC, NPAD, D), jnp.float32),
    scratch_types=[
        pltpu.VMEM((NB, EB), jnp.int32),
        pltpu.VMEM((3, EB), jnp.int32),
        pltpu.VMEM((EB, D), jnp.float32),
        pltpu.VMEM((EB, D), jnp.float32),
        pltpu.VMEM_SHARED((NPAD, D), jnp.float32),
        pltpu.SemaphoreType.DMA,
        pltpu.SemaphoreType.DMA,
    ],
)
def _deg_kernel(src_hbm, dst_hbm, e0_hbm, e1_hbm, zeros_hbm,
                out_hbm,
                srct_v, dstb_v, e0_v, e1_v, sp, isem, ssem):
    c = lax.axis_index("c")
    s = lax.axis_index("s")
    wid = c * NS + s
    rbase = s * ROWS_PER_TEC

    pltpu.sync_copy(zeros_hbm.at[pl.ds(rbase, ROWS_PER_TEC)],
                    sp.at[pl.ds(rbase, ROWS_PER_TEC)])
    pltpu.sync_copy(e0_hbm, e0_v)
    pltpu.sync_copy(e1_hbm, e1_v)
    pltpu.sync_copy(src_hbm.at[wid], srct_v)
    plsc.subcore_barrier()

    pltpu.async_copy(dst_hbm.at[wid, 0], dstb_v.at[0], isem)

    def body(m, carry):
        @pl.when(m + 1 < NB)
        def _():
            pltpu.async_copy(dst_hbm.at[wid, m + 1],
                             dstb_v.at[lax.rem(m + 1, 3)], isem)

        pltpu.async_copy(e0_v, sp.at[srct_v.at[m]], ssem, add=True)
        pltpu.make_async_copy(dst_hbm.at[wid, 0], dstb_v.at[0], isem).wait()
        pltpu.async_copy(e1_v, sp.at[dstb_v.at[lax.rem(m, 3)]], ssem, add=True)

        @pl.when(m >= 1)
        def _():
            pltpu.make_async_copy(e0_v, sp.at[pl.ds(0, EB)], ssem).wait()
            pltpu.make_async_copy(e0_v, sp.at[pl.ds(0, EB)], ssem).wait()

        return carry

    lax.fori_loop(0, NB, body, 0)
    pltpu.make_async_copy(e0_v, sp.at[pl.ds(0, EB)], ssem).wait()
    pltpu.make_async_copy(e0_v, sp.at[pl.ds(0, EB)], ssem).wait()
    plsc.subcore_barrier()

    pltpu.sync_copy(sp.at[pl.ds(rbase, ROWS_PER_TEC)],
                    out_hbm.at[c, pl.ds(rbase, ROWS_PER_TEC)])


@functools.partial(
    pl.kernel,
    mesh=_mesh,
    out_type=jax.ShapeDtypeStruct((NC, NPAD, D), jnp.float32),
    scratch_types=[
        pltpu.VMEM((NB, EB), jnp.int32),
        pltpu.VMEM((3, EB), jnp.int32),
        pltpu.VMEM((2, EB, D), jnp.float32),
        pltpu.VMEM_SHARED((NPAD, D), jnp.float32),
        pltpu.SemaphoreType.DMA,
        pltpu.SemaphoreType.DMA,
        pltpu.SemaphoreType.DMA,
    ],
)
def _agg_kernel(src_hbm, dst_hbm, hw_hbm, zeros_hbm,
                out_hbm,
                srct_v, dstb_v, rows, agg_sp, isem, gsem, ssem):
    c = lax.axis_index("c")
    s = lax.axis_index("s")
    wid = c * NS + s
    rbase = s * ROWS_PER_TEC

    pltpu.sync_copy(zeros_hbm.at[pl.ds(rbase, ROWS_PER_TEC)],
                    agg_sp.at[pl.ds(rbase, ROWS_PER_TEC)])
    pltpu.sync_copy(src_hbm.at[wid], srct_v)
    plsc.subcore_barrier()

    def _drain_idx():
        pltpu.make_async_copy(dst_hbm.at[wid, 0], dstb_v.at[0], isem).wait()

    def _drain_gather():
        pltpu.make_async_copy(hw_hbm.at[pl.ds(0, EB)], rows.at[0], gsem).wait()

    def _drain_scatter():
        pltpu.make_async_copy(rows.at[0], agg_sp.at[pl.ds(0, EB)], ssem).wait()

    pltpu.async_copy(dst_hbm.at[wid, 0], dstb_v.at[0], isem)
    pltpu.async_copy(hw_hbm.at[srct_v.at[0]], rows.at[0], gsem)

    def body(m, carry):
        b = lax.rem(m, 2)

        @pl.when(m + 1 < NB)
        def _():
            pltpu.async_copy(dst_hbm.at[wid, m + 1],
                             dstb_v.at[lax.rem(m + 1, 3)], isem)

        _drain_gather()
        _drain_idx()
        pltpu.async_copy(rows.at[b], agg_sp.at[dstb_v.at[lax.rem(m, 3)]],
                         ssem, add=True)

        @pl.when(m >= 1)
        def _():
            _drain_scatter()

        @pl.when(m + 1 < NB)
        def _():
            pltpu.async_copy(hw_hbm.at[srct_v.at[m + 1]],
                             rows.at[lax.rem(m + 1, 2)], gsem)

        return carry

    lax.fori_loop(0, NB, body, 0)
    _drain_scatter()
    plsc.subcore_barrier()

    pltpu.sync_copy(agg_sp.at[pl.ds(rbase, ROWS_PER_TEC)],
                    out_hbm.at[c, pl.ds(rbase, ROWS_PER_TEC)])


def _norm_body(deg_ref, ns_ref, nd_ref):
    co = deg_ref[0, :, 0:1] + deg_ref[1, :, 0:1]
    ci = deg_ref[0, :, 1:2] + deg_ref[1, :, 1:2]
    ns_ref[:] = lax.rsqrt(co + 1.0)
    nd_ref[:] = lax.rsqrt(ci + 1.0)


def _norms(deg):
    return pl.pallas_call(
        _norm_body,
        out_shape=[
            jax.ShapeDtypeStruct((NPAD, 1), jnp.float32),
            jax.ShapeDtypeStruct((NPAD, 1), jnp.float32),
        ],
    )(deg)


_BLK = 1024


def _pre_body(h_ref, ns_ref, w_ref, o_ref):
    o_ref[:] = jnp.dot(h_ref[:] * ns_ref[:], w_ref[:],
                       preferred_element_type=jnp.float32)


def _pre_matmul(h, ns, w):
    grid = (NPAD // _BLK,)
    return pl.pallas_call(
        _pre_body,
        grid=grid,
        in_specs=[
            pl.BlockSpec((_BLK, D), lambda i: (i, 0)),
            pl.BlockSpec((_BLK, 1), lambda i: (i, 0)),
            pl.BlockSpec((D, D), lambda i: (0, 0)),
        ],
        out_specs=pl.BlockSpec((_BLK, D), lambda i: (i, 0)),
        out_shape=jax.ShapeDtypeStruct((NPAD, D), jnp.float32),
    )(h, ns, w)


def _mid_body(p_ref, hw_ref, nd_ref, b_ref, ns_ref, w_ref, o_ref):
    agg = p_ref[0] + p_ref[1] + hw_ref[:]
    h = jnp.maximum(agg * nd_ref[:] + b_ref[:], 0.0) * ns_ref[:]
    o_ref[:] = jnp.dot(h, w_ref[:], preferred_element_type=jnp.float32)


def _mid_matmul(parts, hw, nd, b, ns, w):
    grid = (NPAD // _BLK,)
    return pl.pallas_call(
        _mid_body,
        grid=grid,
        in_specs=[
            pl.BlockSpec((NC, _BLK, D), lambda i: (0, i, 0)),
            pl.BlockSpec((_BLK, D), lambda i: (i, 0)),
            pl.BlockSpec((_BLK, 1), lambda i: (i, 0)),
            pl.BlockSpec((1, D), lambda i: (0, 0)),
            pl.BlockSpec((_BLK, 1), lambda i: (i, 0)),
            pl.BlockSpec((D, D), lambda i: (0, 0)),
        ],
        out_specs=pl.BlockSpec((_BLK, D), lambda i: (i, 0)),
        out_shape=jax.ShapeDtypeStruct((NPAD, D), jnp.float32),
    )(parts, hw, nd, b, ns, w)


def _fin_body(p_ref, hw_ref, nd_ref, b_ref, o_ref):
    agg = p_ref[0] + p_ref[1] + hw_ref[:]
    o_ref[:] = agg * nd_ref[:] + b_ref[:]


def _fin_combine(parts, hw, nd, b):
    grid = (NPAD // _BLK,)
    return pl.pallas_call(
        _fin_body,
        grid=grid,
        in_specs=[
            pl.BlockSpec((NC, _BLK, D), lambda i: (0, i, 0)),
            pl.BlockSpec((_BLK, D), lambda i: (i, 0)),
            pl.BlockSpec((_BLK, 1), lambda i: (i, 0)),
            pl.BlockSpec((1, D), lambda i: (0, 0)),
        ],
        out_specs=pl.BlockSpec((_BLK, D), lambda i: (i, 0)),
        out_shape=jax.ShapeDtypeStruct((NPAD, D), jnp.float32),
    )(parts, hw, nd, b)


def kernel(features, edge_index, W1, b1, W2, b2, W3, b3):
    src = edge_index[0].astype(jnp.int32)
    dst = edge_index[1].astype(jnp.int32)
    pad = EPAD - E
    fill = jnp.full((pad,), NPAD - 1, jnp.int32)
    srcp = jnp.concatenate([src, fill]).reshape(NW, NB, EB)
    dstp = jnp.concatenate([dst, fill]).reshape(NW, NB, EB)

    featp = jnp.pad(features, ((0, NPAD - N), (0, 0)))
    zerosD = jnp.zeros((NPAD, D), jnp.float32)
    eye2 = jnp.eye(2, D, dtype=jnp.float32)
    e0 = jnp.broadcast_to(eye2[0:1], (EB, D))
    e1 = jnp.broadcast_to(eye2[1:2], (EB, D))

    deg = _deg_kernel(srcp, dstp, e0, e1, zerosD)
    ns, nd = _norms(deg)

    b1r = b1.reshape(1, D)
    b2r = b2.reshape(1, D)
    b3r = b3.reshape(1, D)

    hw = _pre_matmul(featp, ns, W1)
    parts = _agg_kernel(srcp, dstp, hw, zerosD)
    hw = _mid_matmul(parts, hw, nd, b1r, ns, W2)
    parts = _agg_kernel(srcp, dstp, hw, zerosD)
    hw = _mid_matmul(parts, hw, nd, b2r, ns, W2)
    parts = _agg_kernel(srcp, dstp, hw, zerosD)
    hw = _mid_matmul(parts, hw, nd, b2r, ns, W3)
    parts = _agg_kernel(srcp, dstp, hw, zerosD)
    out = _fin_combine(parts, hw, nd, b3r)

    return out[:N]


# speedup vs baseline: 12.3716x; 2.7490x over previous
import functools

import jax
import jax.numpy as jnp
from jax import lax
from jax.experimental import pallas as pl
from jax.experimental.pallas import tpu as pltpu
from jax.experimental.pallas import tpu_sc as plsc

N = 10000
E = 320000
D = 128

NC = 2
NS = 16
NW = NC * NS

NPAD = 10240
ROWS_PER_TEC = NPAD // NS
EB = 128
EPT = 10240
NB = EPT // EB
EPAD = NW * EPT

_mesh = plsc.VectorSubcoreMesh(core_axis_name="c", subcore_axis_name="s")


@functools.partial(
    pl.kernel,
    mesh=_mesh,
    out_type=jax.ShapeDtypeStruct((NC, NPAD, D), jnp.float32),
    scratch_types=[
        pltpu.VMEM((NB, EB), jnp.int32),
        pltpu.VMEM((3, EB), jnp.int32),
        pltpu.VMEM((EB, D), jnp.float32),
        pltpu.VMEM((EB, D), jnp.float32),
        pltpu.VMEM_SHARED((NPAD, D), jnp.float32),
        pltpu.SemaphoreType.DMA,
        pltpu.SemaphoreType.DMA,
    ],
)
def _deg_kernel(src_hbm, dst_hbm, e0_hbm, e1_hbm, zeros_hbm,
                out_hbm,
                srct_v, dstb_v, e0_v, e1_v, sp, isem, ssem):
    c = lax.axis_index("c")
    s = lax.axis_index("s")
    wid = c * NS + s
    rbase = s * ROWS_PER_TEC

    pltpu.sync_copy(zeros_hbm.at[pl.ds(rbase, ROWS_PER_TEC)],
                    sp.at[pl.ds(rbase, ROWS_PER_TEC)])
    pltpu.sync_copy(e0_hbm, e0_v)
    pltpu.sync_copy(e1_hbm, e1_v)
    pltpu.sync_copy(src_hbm.at[wid], srct_v)
    plsc.subcore_barrier()

    pltpu.async_copy(dst_hbm.at[wid, 0], dstb_v.at[0], isem)

    def body(m, carry):
        @pl.when(m + 1 < NB)
        def _():
            pltpu.async_copy(dst_hbm.at[wid, m + 1],
                             dstb_v.at[lax.rem(m + 1, 3)], isem)

        pltpu.async_copy(e0_v, sp.at[srct_v.at[m]], ssem, add=True)
        pltpu.make_async_copy(dst_hbm.at[wid, 0], dstb_v.at[0], isem).wait()
        pltpu.async_copy(e1_v, sp.at[dstb_v.at[lax.rem(m, 3)]], ssem, add=True)

        @pl.when(m >= 1)
        def _():
            pltpu.make_async_copy(e0_v, sp.at[pl.ds(0, EB)], ssem).wait()
            pltpu.make_async_copy(e0_v, sp.at[pl.ds(0, EB)], ssem).wait()

        return carry

    lax.fori_loop(0, NB, body, 0)
    pltpu.make_async_copy(e0_v, sp.at[pl.ds(0, EB)], ssem).wait()
    pltpu.make_async_copy(e0_v, sp.at[pl.ds(0, EB)], ssem).wait()
    plsc.subcore_barrier()

    pltpu.sync_copy(sp.at[pl.ds(rbase, ROWS_PER_TEC)],
                    out_hbm.at[c, pl.ds(rbase, ROWS_PER_TEC)])


@functools.partial(
    pl.kernel,
    mesh=_mesh,
    out_type=jax.ShapeDtypeStruct((NC, NPAD, D), jnp.float32),
    scratch_types=[
        pltpu.VMEM((NB, EB), jnp.int32),
        pltpu.VMEM((3, EB), jnp.int32),
        pltpu.VMEM((2, EB, D), jnp.float32),
        pltpu.VMEM_SHARED((NPAD, D), jnp.float32),
        pltpu.SemaphoreType.DMA,
        pltpu.SemaphoreType.DMA,
        pltpu.SemaphoreType.DMA,
    ],
)
def _agg_kernel(src_hbm, dst_hbm, hw_hbm, zeros_hbm,
                out_hbm,
                srct_v, dstb_v, rows, agg_sp, isem, gsem, ssem):
    c = lax.axis_index("c")
    s = lax.axis_index("s")
    wid = c * NS + s
    rbase = s * ROWS_PER_TEC

    pltpu.sync_copy(zeros_hbm.at[pl.ds(rbase, ROWS_PER_TEC)],
                    agg_sp.at[pl.ds(rbase, ROWS_PER_TEC)])
    pltpu.sync_copy(src_hbm.at[wid], srct_v)
    plsc.subcore_barrier()

    def _drain_idx():
        pltpu.make_async_copy(dst_hbm.at[wid, 0], dstb_v.at[0], isem).wait()

    def _drain_gather():
        pltpu.make_async_copy(hw_hbm.at[pl.ds(0, EB)], rows.at[0], gsem).wait()

    def _drain_scatter():
        pltpu.make_async_copy(rows.at[0], agg_sp.at[pl.ds(0, EB)], ssem).wait()

    pltpu.async_copy(dst_hbm.at[wid, 0], dstb_v.at[0], isem)
    pltpu.async_copy(hw_hbm.at[srct_v.at[0]], rows.at[0], gsem)

    def body(m, carry):
        b = lax.rem(m, 2)

        @pl.when(m + 1 < NB)
        def _():
            pltpu.async_copy(dst_hbm.at[wid, m + 1],
                             dstb_v.at[lax.rem(m + 1, 3)], isem)

        _drain_gather()
        _drain_idx()
        pltpu.async_copy(rows.at[b], agg_sp.at[dstb_v.at[lax.rem(m, 3)]],
                         ssem, add=True)

        @pl.when(m >= 1)
        def _():
            _drain_scatter()

        @pl.when(m + 1 < NB)
        def _():
            pltpu.async_copy(hw_hbm.at[srct_v.at[m + 1]],
                             rows.at[lax.rem(m + 1, 2)], gsem)

        return carry

    lax.fori_loop(0, NB, body, 0)
    _drain_scatter()
    plsc.subcore_barrier()

    pltpu.sync_copy(agg_sp.at[pl.ds(rbase, ROWS_PER_TEC)],
                    out_hbm.at[c, pl.ds(rbase, ROWS_PER_TEC)])


def _norm_body(deg_ref, ns_ref, nd_ref):
    co = deg_ref[0, :, 0:1] + deg_ref[1, :, 0:1]
    ci = deg_ref[0, :, 1:2] + deg_ref[1, :, 1:2]
    ns_ref[:] = lax.rsqrt(co + 1.0)
    nd_ref[:] = lax.rsqrt(ci + 1.0)


def _norms(deg):
    return pl.pallas_call(
        _norm_body,
        out_shape=[
            jax.ShapeDtypeStruct((NPAD, 1), jnp.float32),
            jax.ShapeDtypeStruct((NPAD, 1), jnp.float32),
        ],
    )(deg)


_BLK = 1024


def _pre_body(h_ref, ns_ref, w_ref, o_ref):
    o_ref[:] = jnp.dot(h_ref[:] * ns_ref[:], w_ref[:],
                       preferred_element_type=jnp.float32)


def _pre_matmul(h, ns, w):
    grid = (NPAD // _BLK,)
    return pl.pallas_call(
        _pre_body,
        grid=grid,
        in_specs=[
            pl.BlockSpec((_BLK, D), lambda i: (i, 0)),
            pl.BlockSpec((_BLK, 1), lambda i: (i, 0)),
            pl.BlockSpec((D, D), lambda i: (0, 0)),
        ],
        out_specs=pl.BlockSpec((_BLK, D), lambda i: (i, 0)),
        out_shape=jax.ShapeDtypeStruct((NPAD, D), jnp.float32),
    )(h, ns, w)


def _mid_body(p_ref, hw_ref, nd_ref, b_ref, ns_ref, w_ref, o_ref):
    agg = p_ref[0] + p_ref[1] + hw_ref[:]
    h = jnp.maximum(agg * nd_ref[:] + b_ref[:], 0.0) * ns_ref[:]
    o_ref[:] = jnp.dot(h, w_ref[:], preferred_element_type=jnp.float32)


def _mid_matmul(parts, hw, nd, b, ns, w):
    grid = (NPAD // _BLK,)
    return pl.pallas_call(
        _mid_body,
        grid=grid,
        in_specs=[
            pl.BlockSpec((NC, _BLK, D), lambda i: (0, i, 0)),
            pl.BlockSpec((_BLK, D), lambda i: (i, 0)),
            pl.BlockSpec((_BLK, 1), lambda i: (i, 0)),
            pl.BlockSpec((1, D), lambda i: (0, 0)),
            pl.BlockSpec((_BLK, 1), lambda i: (i, 0)),
            pl.BlockSpec((D, D), lambda i: (0, 0)),
        ],
        out_specs=pl.BlockSpec((_BLK, D), lambda i: (i, 0)),
        out_shape=jax.ShapeDtypeStruct((NPAD, D), jnp.float32),
    )(parts, hw, nd, b, ns, w)


def _fin_body(p_ref, hw_ref, nd_ref, b_ref, o_ref):
    agg = p_ref[0] + p_ref[1] + hw_ref[:]
    o_ref[:] = agg * nd_ref[:] + b_ref[:]


def _fin_combine(parts, hw, nd, b):
    grid = (NPAD // _BLK,)
    return pl.pallas_call(
        _fin_body,
        grid=grid,
        in_specs=[
            pl.BlockSpec((NC, _BLK, D), lambda i: (0, i, 0)),
            pl.BlockSpec((_BLK, D), lambda i: (i, 0)),
            pl.BlockSpec((_BLK, 1), lambda i: (i, 0)),
            pl.BlockSpec((1, D), lambda i: (0, 0)),
        ],
        out_specs=pl.BlockSpec((_BLK, D), lambda i: (i, 0)),
        out_shape=jax.ShapeDtypeStruct((NPAD, D), jnp.float32),
    )(parts, hw, nd, b)


def kernel(features, edge_index, W1, b1, W2, b2, W3, b3):
    src = edge_index[0].astype(jnp.int32)
    dst = edge_index[1].astype(jnp.int32)
    pad = EPAD - E
    fill = N + (jnp.arange(pad, dtype=jnp.int32) % (NPAD - N))
    srcp = jnp.concatenate([src, fill]).reshape(NW, NB, EB)
    dstp = jnp.concatenate([dst, fill]).reshape(NW, NB, EB)

    featp = jnp.pad(features, ((0, NPAD - N), (0, 0)))
    zerosD = jnp.zeros((NPAD, D), jnp.float32)
    eye2 = jnp.eye(2, D, dtype=jnp.float32)
    e0 = jnp.broadcast_to(eye2[0:1], (EB, D))
    e1 = jnp.broadcast_to(eye2[1:2], (EB, D))

    deg = _deg_kernel(srcp, dstp, e0, e1, zerosD)
    ns, nd = _norms(deg)

    b1r = b1.reshape(1, D)
    b2r = b2.reshape(1, D)
    b3r = b3.reshape(1, D)

    hw = _pre_matmul(featp, ns, W1)
    parts = _agg_kernel(srcp, dstp, hw, zerosD)
    hw = _mid_matmul(parts, hw, nd, b1r, ns, W2)
    parts = _agg_kernel(srcp, dstp, hw, zerosD)
    hw = _mid_matmul(parts, hw, nd, b2r, ns, W2)
    parts = _agg_kernel(srcp, dstp, hw, zerosD)
    hw = _mid_matmul(parts, hw, nd, b2r, ns, W3)
    parts = _agg_kernel(srcp, dstp, hw, zerosD)
    out = _fin_combine(parts, hw, nd, b3r)

    return out[:N]


# speedup vs baseline: 15.4708x; 1.2505x over previous
import functools

import jax
import jax.numpy as jnp
from jax import lax
from jax.experimental import pallas as pl
from jax.experimental.pallas import tpu as pltpu
from jax.experimental.pallas import tpu_sc as plsc

N = 10000
E = 320000
D = 128

NC = 2
NS = 16
NW = NC * NS

NPAD = 10112
ROWS_PER_TEC = NPAD // NS
EB = 128
EPT = 10240
NB = EPT // EB
EPAD = NW * EPT

_mesh = plsc.VectorSubcoreMesh(core_axis_name="c", subcore_axis_name="s")


@functools.partial(
    pl.kernel,
    mesh=_mesh,
    out_type=jax.ShapeDtypeStruct((NC, NPAD, D), jnp.float32),
    scratch_types=[
        pltpu.VMEM((NB, EB), jnp.int32),
        pltpu.VMEM((3, EB), jnp.int32),
        pltpu.VMEM((EB, D), jnp.float32),
        pltpu.VMEM((EB, D), jnp.float32),
        pltpu.VMEM_SHARED((NPAD, D), jnp.float32),
        pltpu.SemaphoreType.DMA,
        pltpu.SemaphoreType.DMA,
    ],
)
def _deg_kernel(src_hbm, dst_hbm, e0_hbm, e1_hbm, zeros_hbm,
                out_hbm,
                srct_v, dstb_v, e0_v, e1_v, sp, isem, ssem):
    c = lax.axis_index("c")
    s = lax.axis_index("s")
    wid = c * NS + s
    rbase = s * ROWS_PER_TEC

    pltpu.sync_copy(zeros_hbm.at[pl.ds(rbase, ROWS_PER_TEC)],
                    sp.at[pl.ds(rbase, ROWS_PER_TEC)])
    pltpu.sync_copy(e0_hbm, e0_v)
    pltpu.sync_copy(e1_hbm, e1_v)
    pltpu.sync_copy(src_hbm.at[wid], srct_v)
    plsc.subcore_barrier()

    pltpu.async_copy(dst_hbm.at[wid, 0], dstb_v.at[0], isem)

    def body(m, carry):
        @pl.when(m + 1 < NB)
        def _():
            pltpu.async_copy(dst_hbm.at[wid, m + 1],
                             dstb_v.at[lax.rem(m + 1, 3)], isem)

        pltpu.async_copy(e0_v, sp.at[srct_v.at[m]], ssem, add=True)
        pltpu.make_async_copy(dst_hbm.at[wid, 0], dstb_v.at[0], isem).wait()
        pltpu.async_copy(e1_v, sp.at[dstb_v.at[lax.rem(m, 3)]], ssem, add=True)

        @pl.when(m >= 1)
        def _():
            pltpu.make_async_copy(e0_v, sp.at[pl.ds(0, EB)], ssem).wait()
            pltpu.make_async_copy(e0_v, sp.at[pl.ds(0, EB)], ssem).wait()

        return carry

    lax.fori_loop(0, NB, body, 0)
    pltpu.make_async_copy(e0_v, sp.at[pl.ds(0, EB)], ssem).wait()
    pltpu.make_async_copy(e0_v, sp.at[pl.ds(0, EB)], ssem).wait()
    plsc.subcore_barrier()

    pltpu.sync_copy(sp.at[pl.ds(rbase, ROWS_PER_TEC)],
                    out_hbm.at[c, pl.ds(rbase, ROWS_PER_TEC)])


@functools.partial(
    pl.kernel,
    mesh=_mesh,
    out_type=jax.ShapeDtypeStruct((NC, NPAD, D), jnp.float32),
    scratch_types=[
        pltpu.VMEM((3, EB), jnp.int32),
        pltpu.VMEM((4, EB), jnp.int32),
        pltpu.VMEM((3, EB, D), jnp.float32),
        pltpu.VMEM_SHARED((NPAD, D), jnp.float32),
        pltpu.SemaphoreType.DMA,
        pltpu.SemaphoreType.DMA,
        pltpu.SemaphoreType.DMA,
    ],
)
def _agg_kernel(src_hbm, dst_hbm, hw_hbm, zeros_hbm,
                out_hbm,
                srcb_v, dstb_v, rows, agg_sp, isem, gsem, ssem):
    c = lax.axis_index("c")
    s = lax.axis_index("s")
    wid = c * NS + s
    rbase = s * ROWS_PER_TEC

    pltpu.sync_copy(zeros_hbm.at[pl.ds(rbase, ROWS_PER_TEC)],
                    agg_sp.at[pl.ds(rbase, ROWS_PER_TEC)])
    plsc.subcore_barrier()

    def _fire_pair(m):
        pltpu.async_copy(src_hbm.at[wid, m], srcb_v.at[lax.rem(m, 3)], isem)
        pltpu.async_copy(dst_hbm.at[wid, m], dstb_v.at[lax.rem(m, 4)], isem)

    def _drain_pair():
        pltpu.make_async_copy(src_hbm.at[wid, 0], srcb_v.at[0], isem).wait()
        pltpu.make_async_copy(src_hbm.at[wid, 0], srcb_v.at[0], isem).wait()

    def _fire_gather(m):
        pltpu.async_copy(hw_hbm.at[srcb_v.at[lax.rem(m, 3)]],
                         rows.at[lax.rem(m, 3)], gsem)

    def _drain_gather():
        pltpu.make_async_copy(hw_hbm.at[pl.ds(0, EB)], rows.at[0], gsem).wait()

    def _drain_scatter():
        pltpu.make_async_copy(rows.at[0], agg_sp.at[pl.ds(0, EB)], ssem).wait()

    def prime(m, carry):
        _fire_pair(m)
        return carry

    lax.fori_loop(0, 3, prime, 0)
    _drain_pair()
    _fire_gather(0)
    _drain_pair()
    _fire_gather(1)

    def body(m, carry):
        @pl.when(m >= 1)
        def _():
            _drain_scatter()

        _drain_gather()
        pltpu.async_copy(rows.at[lax.rem(m, 3)],
                         agg_sp.at[dstb_v.at[lax.rem(m, 4)]], ssem, add=True)

        @pl.when(m + 3 < NB)
        def _():
            _fire_pair(m + 3)

        @pl.when(m + 2 < NB)
        def _():
            _drain_pair()
            _fire_gather(m + 2)

        return carry

    lax.fori_loop(0, NB, body, 0)
    _drain_scatter()
    plsc.subcore_barrier()

    pltpu.sync_copy(agg_sp.at[pl.ds(rbase, ROWS_PER_TEC)],
                    out_hbm.at[c, pl.ds(rbase, ROWS_PER_TEC)])


def _pre_body(deg_ref, h_ref, w_ref, hw_ref, ns_ref, nd_ref):
    co = deg_ref[0, :, 0:1] + deg_ref[1, :, 0:1]
    ci = deg_ref[0, :, 1:2] + deg_ref[1, :, 1:2]
    ns = lax.rsqrt(co + 1.0)
    ns_ref[:] = ns
    nd_ref[:] = lax.rsqrt(ci + 1.0)
    hw_ref[:] = jnp.dot(h_ref[:] * ns, w_ref[:],
                        preferred_element_type=jnp.float32)


def _pre_matmul(deg, h, w):
    return pl.pallas_call(
        _pre_body,
        out_shape=[
            jax.ShapeDtypeStruct((NPAD, D), jnp.float32),
            jax.ShapeDtypeStruct((NPAD, 1), jnp.float32),
            jax.ShapeDtypeStruct((NPAD, 1), jnp.float32),
        ],
    )(deg, h, w)


def _mid_body(p_ref, hw_ref, nd_ref, b_ref, ns_ref, w_ref, o_ref):
    agg = p_ref[0] + p_ref[1] + hw_ref[:]
    h = jnp.maximum(agg * nd_ref[:] + b_ref[:], 0.0) * ns_ref[:]
    o_ref[:] = jnp.dot(h, w_ref[:], preferred_element_type=jnp.float32)


def _mid_matmul(parts, hw, nd, b, ns, w):
    return pl.pallas_call(
        _mid_body,
        out_shape=jax.ShapeDtypeStruct((NPAD, D), jnp.float32),
    )(parts, hw, nd, b, ns, w)


def _fin_body(p_ref, hw_ref, nd_ref, b_ref, o_ref):
    agg = p_ref[0] + p_ref[1] + hw_ref[:]
    o_ref[:] = agg * nd_ref[:] + b_ref[:]


def _fin_combine(parts, hw, nd, b):
    return pl.pallas_call(
        _fin_body,
        out_shape=jax.ShapeDtypeStruct((NPAD, D), jnp.float32),
    )(parts, hw, nd, b)


def kernel(features, edge_index, W1, b1, W2, b2, W3, b3):
    src = edge_index[0].astype(jnp.int32)
    dst = edge_index[1].astype(jnp.int32)
    pad = EPAD - E
    fill = N + (jnp.arange(pad, dtype=jnp.int32) % (NPAD - N))
    srcp = jnp.concatenate([src, fill]).reshape(NW, NB, EB)
    dstp = jnp.concatenate([dst, fill]).reshape(NW, NB, EB)

    featp = jnp.pad(features, ((0, NPAD - N), (0, 0)))
    zerosD = jnp.zeros((NPAD, D), jnp.float32)
    eye2 = jnp.eye(2, D, dtype=jnp.float32)
    e0 = jnp.broadcast_to(eye2[0:1], (EB, D))
    e1 = jnp.broadcast_to(eye2[1:2], (EB, D))

    deg = _deg_kernel(srcp, dstp, e0, e1, zerosD)

    b1r = b1.reshape(1, D)
    b2r = b2.reshape(1, D)
    b3r = b3.reshape(1, D)

    hw, ns, nd = _pre_matmul(deg, featp, W1)
    parts = _agg_kernel(srcp, dstp, hw, zerosD)
    hw = _mid_matmul(parts, hw, nd, b1r, ns, W2)
    parts = _agg_kernel(srcp, dstp, hw, zerosD)
    hw = _mid_matmul(parts, hw, nd, b2r, ns, W2)
    parts = _agg_kernel(srcp, dstp, hw, zerosD)
    hw = _mid_matmul(parts, hw, nd, b2r, ns, W3)
    parts = _agg_kernel(srcp, dstp, hw, zerosD)
    out = _fin_combine(parts, hw, nd, b3r)

    return out[:N]
